# Initial kernel scaffold; baseline (speedup 1.0000x reference)
import functools

import jax
import jax.numpy as jnp
from jax import lax
from jax.experimental import pallas as pl
from jax.experimental.pallas import tpu as pltpu
from jax.experimental.pallas import tpu_sc as plsc

N = 10000
E = 320000
D = 128

NC = 2
NS = 16
NW = NC * NS
CH = 128
CPW = (E // CH + NW - 1) // NW
E_PAD = CPW * NW * CH
ROWS_PER_TILE = N // NS
ACC_ROWS = N + 8


def _seg_sum_sc(m, src2d, dst2d):
    mesh = plsc.VectorSubcoreMesh(core_axis_name="c", subcore_axis_name="s")

    @functools.partial(
        pl.kernel,
        out_type=(
            jax.ShapeDtypeStruct((N, D), jnp.float32),
            jax.ShapeDtypeStruct((N, D), jnp.float32),
        ),
        mesh=mesh,
        scratch_types=[
            pltpu.VMEM((CPW, CH), jnp.int32),
            pltpu.VMEM((CPW, CH), jnp.int32),
            pltpu.VMEM((CH, D), jnp.float32),
            pltpu.VMEM_SHARED((ACC_ROWS, D), jnp.float32),
            pltpu.SemaphoreType.DMA,
        ],
    )
    def seg_kernel(m_hbm, src_hbm, dst_hbm, p0_hbm, p1_hbm,
                   src_v, dst_v, rows_v, acc_sh, sem):
        c = lax.axis_index("c")
        s = lax.axis_index("s")
        wid = c * NS + s

        pltpu.sync_copy(src_hbm.at[pl.ds(wid * CPW, CPW)], src_v)
        pltpu.sync_copy(dst_hbm.at[pl.ds(wid * CPW, CPW)], dst_v)

        @pl.loop(0, CH)
        def _zero_rows(i):
            @pl.loop(0, D // 16)
            def _zero_lanes(k16):
                rows_v[i, pl.ds(k16 * 16, 16)] = jnp.zeros((16,), jnp.float32)

        base = s * ROWS_PER_TILE

        @pl.loop(0, ROWS_PER_TILE // CH)
        def _zero_acc(t):
            pltpu.sync_copy(rows_v, acc_sh.at[pl.ds(base + t * CH, CH)])

        rem = ROWS_PER_TILE % CH
        if rem:
            pltpu.sync_copy(
                rows_v.at[pl.ds(0, rem)],
                acc_sh.at[pl.ds(base + (ROWS_PER_TILE // CH) * CH, rem)])
        plsc.subcore_barrier()

        @pl.loop(0, CPW)
        def _edges(i):
            pltpu.async_copy(m_hbm.at[src_v.at[i]], rows_v, sem).wait()
            pltpu.sync_copy(rows_v, acc_sh.at[dst_v.at[i]], add=True)
        plsc.subcore_barrier()

        @pl.when(c == 0)
        def _out0():
            pltpu.sync_copy(acc_sh.at[pl.ds(base, ROWS_PER_TILE)],
                            p0_hbm.at[pl.ds(base, ROWS_PER_TILE)])

        @pl.when(c == 1)
        def _out1():
            pltpu.sync_copy(acc_sh.at[pl.ds(base, ROWS_PER_TILE)],
                            p1_hbm.at[pl.ds(base, ROWS_PER_TILE)])

    return seg_kernel(m, src2d, dst2d)


def _tc_pre(x, W_rel, W_root, b):
    def body(x_ref, wr_ref, wo_ref, b_ref, m_ref, r_ref):
        dn = (((1,), (1,)), ((), ()))
        xx = x_ref[...]
        m_ref[...] = lax.dot_general(xx, wr_ref[...], dn,
                                     preferred_element_type=jnp.float32,
                                     precision=lax.Precision.HIGHEST)
        r_ref[...] = lax.dot_general(xx, wo_ref[...], dn,
                                     preferred_element_type=jnp.float32,
                                     precision=lax.Precision.HIGHEST) + b_ref[...]

    return pl.pallas_call(
        body,
        out_shape=(jax.ShapeDtypeStruct((N, D), jnp.float32),
                   jax.ShapeDtypeStruct((N, D), jnp.float32)),
    )(x, W_rel, W_root, b.reshape(1, D))


def _tc_mid(p0, p1, r, W_rel, W_root, b):
    def body(p0_ref, p1_ref, r_ref, wr_ref, wo_ref, b_ref, m_ref, r2_ref):
        h = jnp.maximum(p0_ref[...] + p1_ref[...] + r_ref[...], 0.0)
        dn = (((1,), (1,)), ((), ()))
        m_ref[...] = lax.dot_general(h, wr_ref[...], dn,
                                     preferred_element_type=jnp.float32,
                                     precision=lax.Precision.HIGHEST)
        r2_ref[...] = lax.dot_general(h, wo_ref[...], dn,
                                      preferred_element_type=jnp.float32,
                                      precision=lax.Precision.HIGHEST) + b_ref[...]

    return pl.pallas_call(
        body,
        out_shape=(jax.ShapeDtypeStruct((N, D), jnp.float32),
                   jax.ShapeDtypeStruct((N, D), jnp.float32)),
    )(p0, p1, r, W_rel, W_root, b.reshape(1, D))


def _tc_final(p0, p1, r):
    def body(p0_ref, p1_ref, r_ref, o_ref):
        o_ref[...] = p0_ref[...] + p1_ref[...] + r_ref[...]

    return pl.pallas_call(
        body,
        out_shape=jax.ShapeDtypeStruct((N, D), jnp.float32),
    )(p0, p1, r)


def kernel(x, relationsedge_indices_relations, W_rel1, b_rel1, W_root1,
           W_rel2, b_rel2, W_root2):
    ei = relationsedge_indices_relations[-1]
    src, dst = ei[0], ei[1]
    pad = E_PAD - E
    src2d = jnp.concatenate(
        [src, jnp.zeros((pad,), jnp.int32)]).reshape(NW * CPW, CH)
    dst2d = jnp.concatenate(
        [dst, jnp.full((pad,), N, jnp.int32)]).reshape(NW * CPW, CH)

    m1, r1 = _tc_pre(x, W_rel1, W_root1, b_rel1)
    p0, p1 = _seg_sum_sc(m1, src2d, dst2d)
    m2, r2 = _tc_mid(p0, p1, r1, W_rel2, W_root2, b_rel2)
    q0, q1 = _seg_sum_sc(m2, src2d, dst2d)
    out = _tc_final(q0, q1, r2)
    return out.reshape(N, 1, D)


# speedup vs baseline: 2.7989x; 2.7989x over previous
import functools

import jax
import jax.numpy as jnp
from jax import lax
from jax.experimental import pallas as pl
from jax.experimental.pallas import tpu as pltpu
from jax.experimental.pallas import tpu_sc as plsc

N = 10000
E = 320000
D = 128

NC = 2
NS = 16
NW = NC * NS
CH = 128
CPW = 80
E_PAD = CPW * NW * CH
TILE_ROWS = 632
LAST_ROWS = N - 15 * TILE_ROWS
ACC_ROWS = NS * TILE_ROWS


def _seg_sum_sc(m, src2d, dst2d):
    mesh = plsc.VectorSubcoreMesh(core_axis_name="c", subcore_axis_name="s")

    @functools.partial(
        pl.kernel,
        out_type=(
            jax.ShapeDtypeStruct((N, D), jnp.float32),
            jax.ShapeDtypeStruct((N, D), jnp.float32),
        ),
        mesh=mesh,
        scratch_types=[
            pltpu.VMEM((CPW, CH), jnp.int32),
            pltpu.VMEM((CPW, CH), jnp.int32),
            pltpu.VMEM((CH, D), jnp.float32),
            pltpu.VMEM_SHARED((ACC_ROWS, D), jnp.float32),
            pltpu.SemaphoreType.DMA,
        ],
    )
    def seg_kernel(m_hbm, src_hbm, dst_hbm, p0_hbm, p1_hbm,
                   src_v, dst_v, rows_v, acc_sh, sem):
        c = lax.axis_index("c")
        s = lax.axis_index("s")
        wid = c * NS + s

        pltpu.sync_copy(src_hbm.at[pl.ds(wid * CPW, CPW)], src_v)
        pltpu.sync_copy(dst_hbm.at[pl.ds(wid * CPW, CPW)], dst_v)

        @pl.loop(0, CH)
        def _zero_rows(i):
            @pl.loop(0, D // 16)
            def _zero_lanes(k16):
                rows_v[i, pl.ds(k16 * 16, 16)] = jnp.zeros((16,), jnp.float32)

        base = s * TILE_ROWS

        @pl.loop(0, TILE_ROWS // CH)
        def _zero_acc(t):
            pltpu.sync_copy(rows_v, acc_sh.at[pl.ds(base + t * CH, CH)])

        rem = TILE_ROWS % CH
        if rem:
            pltpu.sync_copy(
                rows_v.at[pl.ds(0, rem)],
                acc_sh.at[pl.ds(base + (TILE_ROWS // CH) * CH, rem)])
        plsc.subcore_barrier()

        @pl.loop(0, CPW)
        def _edges(i):
            pltpu.async_copy(m_hbm.at[src_v.at[i]], rows_v, sem).wait()
            pltpu.sync_copy(rows_v, acc_sh.at[dst_v.at[i]], add=True)
        plsc.subcore_barrier()

        @pl.when(jnp.logical_and(c == 0, s < NS - 1))
        def _out0():
            pltpu.sync_copy(acc_sh.at[pl.ds(base, TILE_ROWS)],
                            p0_hbm.at[pl.ds(base, TILE_ROWS)])

        @pl.when(jnp.logical_and(c == 0, s == NS - 1))
        def _out0_last():
            pltpu.sync_copy(acc_sh.at[pl.ds(base, LAST_ROWS)],
                            p0_hbm.at[pl.ds(base, LAST_ROWS)])

        @pl.when(jnp.logical_and(c == 1, s < NS - 1))
        def _out1():
            pltpu.sync_copy(acc_sh.at[pl.ds(base, TILE_ROWS)],
                            p1_hbm.at[pl.ds(base, TILE_ROWS)])

        @pl.when(jnp.logical_and(c == 1, s == NS - 1))
        def _out1_last():
            pltpu.sync_copy(acc_sh.at[pl.ds(base, LAST_ROWS)],
                            p1_hbm.at[pl.ds(base, LAST_ROWS)])

    return seg_kernel(m, src2d, dst2d)


def _tc_pre(x, W_rel, W_root, b):
    def body(x_ref, wr_ref, wo_ref, b_ref, m_ref, r_ref):
        dn = (((1,), (1,)), ((), ()))
        xx = x_ref[...]
        m_ref[...] = lax.dot_general(xx, wr_ref[...], dn,
                                     preferred_element_type=jnp.float32,
                                     precision=lax.Precision.HIGHEST)
        r_ref[...] = lax.dot_general(xx, wo_ref[...], dn,
                                     preferred_element_type=jnp.float32,
                                     precision=lax.Precision.HIGHEST) + b_ref[...]

    return pl.pallas_call(
        body,
        out_shape=(jax.ShapeDtypeStruct((N, D), jnp.float32),
                   jax.ShapeDtypeStruct((N, D), jnp.float32)),
    )(x, W_rel, W_root, b.reshape(1, D))


def _tc_mid(p0, p1, r, W_rel, W_root, b):
    def body(p0_ref, p1_ref, r_ref, wr_ref, wo_ref, b_ref, m_ref, r2_ref):
        h = jnp.maximum(p0_ref[...] + p1_ref[...] + r_ref[...], 0.0)
        dn = (((1,), (1,)), ((), ()))
        m_ref[...] = lax.dot_general(h, wr_ref[...], dn,
                                     preferred_element_type=jnp.float32,
                                     precision=lax.Precision.HIGHEST)
        r2_ref[...] = lax.dot_general(h, wo_ref[...], dn,
                                      preferred_element_type=jnp.float32,
                                      precision=lax.Precision.HIGHEST) + b_ref[...]

    return pl.pallas_call(
        body,
        out_shape=(jax.ShapeDtypeStruct((N, D), jnp.float32),
                   jax.ShapeDtypeStruct((N, D), jnp.float32)),
    )(p0, p1, r, W_rel, W_root, b.reshape(1, D))


def _tc_final(p0, p1, r):
    def body(p0_ref, p1_ref, r_ref, o_ref):
        o_ref[...] = p0_ref[...] + p1_ref[...] + r_ref[...]

    return pl.pallas_call(
        body,
        out_shape=jax.ShapeDtypeStruct((N, D), jnp.float32),
    )(p0, p1, r)


def kernel(x, relationsedge_indices_relations, W_rel1, b_rel1, W_root1,
           W_rel2, b_rel2, W_root2):
    ei = relationsedge_indices_relations[-1]
    src, dst = ei[0], ei[1]
    pad = E_PAD - E
    src2d = jnp.concatenate(
        [src, jnp.zeros((pad,), jnp.int32)]).reshape(NW * CPW, CH)
    dst2d = jnp.concatenate(
        [dst, jnp.full((pad,), N, jnp.int32)]).reshape(NW * CPW, CH)

    m1, r1 = _tc_pre(x, W_rel1, W_root1, b_rel1)
    p0, p1 = _seg_sum_sc(m1, src2d, dst2d)
    m2, r2 = _tc_mid(p0, p1, r1, W_rel2, W_root2, b_rel2)
    q0, q1 = _seg_sum_sc(m2, src2d, dst2d)
    out = _tc_final(q0, q1, r2)
    return out.reshape(N, 1, D)


# speedup vs baseline: 3.4008x; 1.2151x over previous
import functools

import jax
import jax.numpy as jnp
from jax import lax
from jax.experimental import pallas as pl
from jax.experimental.pallas import tpu as pltpu
from jax.experimental.pallas import tpu_sc as plsc

N = 10000
E = 320000
D = 128

NC = 2
NS = 16
NW = NC * NS
CH = 128
CPW = 80
NCHUNK = NW * CPW
E_PAD = NCHUNK * CH
M_ROWS = N + 8
TILE_ROWS = 632
LAST_ROWS = N - (NS - 1) * TILE_ROWS
NB = 2


def _seg_sum_sc(m, src2d, dst3d):
    mesh = plsc.VectorSubcoreMesh(core_axis_name="c", subcore_axis_name="s")

    @functools.partial(
        pl.kernel,
        out_type=(
            jax.ShapeDtypeStruct((N, D), jnp.float32),
            jax.ShapeDtypeStruct((N, D), jnp.float32),
        ),
        mesh=mesh,
        scratch_types=[
            pltpu.VMEM((CPW, CH), jnp.int32),
            pltpu.VMEM((NB, CH), jnp.int32),
            pltpu.VMEM((NB, CH, D), jnp.float32),
            pltpu.VMEM_SHARED((N, D), jnp.float32),
        ] + [pltpu.SemaphoreType.DMA] * (2 * NB),
    )
    def seg_kernel(m_hbm, src_hbm, dst_hbm, p0_hbm, p1_hbm,
                   src_v, dst_v, rows_v, acc_sh, *sems):
        semg = sems[:NB]
        semd = sems[NB:]
        c = lax.axis_index("c")
        s = lax.axis_index("s")
        wid = c * NS + s
        cbase = wid * CPW

        pltpu.sync_copy(src_hbm.at[pl.ds(cbase, CPW)], src_v)

        @pl.loop(0, CH)
        def _zero_rows(i):
            @pl.loop(0, D // 16)
            def _zero_lanes(k16):
                rows_v[0, i, pl.ds(k16 * 16, 16)] = jnp.zeros((16,), jnp.float32)

        base = s * TILE_ROWS
        nfull = TILE_ROWS // CH
        rem = TILE_ROWS % CH

        @pl.when(s < NS - 1)
        def _zero_full_tile():
            @pl.loop(0, nfull)
            def _zero_acc(t):
                pltpu.sync_copy(rows_v.at[0],
                                acc_sh.at[pl.ds(base + t * CH, CH)])
            pltpu.sync_copy(
                rows_v.at[0].at[pl.ds(0, rem)],
                acc_sh.at[pl.ds(base + nfull * CH, rem)])

        @pl.when(s == NS - 1)
        def _zero_last_tile():
            @pl.loop(0, LAST_ROWS // CH)
            def _zero_acc(t):
                pltpu.sync_copy(rows_v.at[0],
                                acc_sh.at[pl.ds(base + t * CH, CH)])
            pltpu.sync_copy(
                rows_v.at[0].at[pl.ds(0, LAST_ROWS % CH)],
                acc_sh.at[pl.ds(base + (LAST_ROWS // CH) * CH,
                                LAST_ROWS % CH)])

        for b in range(NB):
            pltpu.async_copy(dst_hbm.at[cbase + b].at[0], dst_v.at[b], semd[b])
            pltpu.async_copy(m_hbm.at[src_v.at[b]], rows_v.at[b], semg[b])
        plsc.subcore_barrier()

        @pl.loop(0, CPW, step=NB)
        def _edges(t):
            for b in range(NB):
                cur = t + b
                pltpu.make_async_copy(
                    m_hbm.at[src_v.at[cur]], rows_v.at[b], semg[b]).wait()
                pltpu.make_async_copy(
                    dst_hbm.at[cbase + cur].at[0], dst_v.at[b], semd[b]).wait()
                pltpu.sync_copy(rows_v.at[b], acc_sh.at[dst_v.at[b]],
                                add=True)
                nxt = cur + NB

                @pl.when(nxt < CPW)
                def _prefetch():
                    pltpu.async_copy(
                        dst_hbm.at[cbase + nxt].at[0], dst_v.at[b], semd[b])
                    pltpu.async_copy(
                        m_hbm.at[src_v.at[nxt]], rows_v.at[b], semg[b])
        plsc.subcore_barrier()

        @pl.when(jnp.logical_and(c == 0, s < NS - 1))
        def _out0():
            pltpu.sync_copy(acc_sh.at[pl.ds(base, TILE_ROWS)],
                            p0_hbm.at[pl.ds(base, TILE_ROWS)])

        @pl.when(jnp.logical_and(c == 0, s == NS - 1))
        def _out0_last():
            pltpu.sync_copy(acc_sh.at[pl.ds(base, LAST_ROWS)],
                            p0_hbm.at[pl.ds(base, LAST_ROWS)])

        @pl.when(jnp.logical_and(c == 1, s < NS - 1))
        def _out1():
            pltpu.sync_copy(acc_sh.at[pl.ds(base, TILE_ROWS)],
                            p1_hbm.at[pl.ds(base, TILE_ROWS)])

        @pl.when(jnp.logical_and(c == 1, s == NS - 1))
        def _out1_last():
            pltpu.sync_copy(acc_sh.at[pl.ds(base, LAST_ROWS)],
                            p1_hbm.at[pl.ds(base, LAST_ROWS)])

    return seg_kernel(m, src2d, dst3d)


def _tc_pre(x, W_rel, W_root, b):
    def body(x_ref, wr_ref, wo_ref, b_ref, m_ref, r_ref):
        dn = (((1,), (1,)), ((), ()))
        xx = x_ref[...]
        m_ref[pl.ds(0, N), :] = lax.dot_general(
            xx, wr_ref[...], dn, preferred_element_type=jnp.float32,
            precision=lax.Precision.HIGHEST)
        m_ref[pl.ds(N, M_ROWS - N), :] = jnp.zeros((M_ROWS - N, D),
                                                   jnp.float32)
        r_ref[...] = lax.dot_general(
            xx, wo_ref[...], dn, preferred_element_type=jnp.float32,
            precision=lax.Precision.HIGHEST) + b_ref[...]

    return pl.pallas_call(
        body,
        out_shape=(jax.ShapeDtypeStruct((M_ROWS, D), jnp.float32),
                   jax.ShapeDtypeStruct((N, D), jnp.float32)),
    )(x, W_rel, W_root, b.reshape(1, D))


def _tc_mid(p0, p1, r, W_rel, W_root, b):
    def body(p0_ref, p1_ref, r_ref, wr_ref, wo_ref, b_ref, m_ref, r2_ref):
        h = jnp.maximum(p0_ref[...] + p1_ref[...] + r_ref[...], 0.0)
        dn = (((1,), (1,)), ((), ()))
        m_ref[pl.ds(0, N), :] = lax.dot_general(
            h, wr_ref[...], dn, preferred_element_type=jnp.float32,
            precision=lax.Precision.HIGHEST)
        m_ref[pl.ds(N, M_ROWS - N), :] = jnp.zeros((M_ROWS - N, D),
                                                   jnp.float32)
        r2_ref[...] = lax.dot_general(
            h, wo_ref[...], dn, preferred_element_type=jnp.float32,
            precision=lax.Precision.HIGHEST) + b_ref[...]

    return pl.pallas_call(
        body,
        out_shape=(jax.ShapeDtypeStruct((M_ROWS, D), jnp.float32),
                   jax.ShapeDtypeStruct((N, D), jnp.float32)),
    )(p0, p1, r, W_rel, W_root, b.reshape(1, D))


def _tc_final(p0, p1, r):
    def body(p0_ref, p1_ref, r_ref, o_ref):
        o_ref[...] = p0_ref[...] + p1_ref[...] + r_ref[...]

    return pl.pallas_call(
        body,
        out_shape=jax.ShapeDtypeStruct((N, D), jnp.float32),
    )(p0, p1, r)


def kernel(x, relationsedge_indices_relations, W_rel1, b_rel1, W_root1,
           W_rel2, b_rel2, W_root2):
    ei = relationsedge_indices_relations[-1]
    src, dst = ei[0], ei[1]
    pad = E_PAD - E
    src2d = jnp.concatenate(
        [src, jnp.full((pad,), N, jnp.int32)]).reshape(NCHUNK, CH)
    dst3d = jnp.concatenate(
        [dst, jnp.zeros((pad,), jnp.int32)]).reshape(NCHUNK, 1, CH)

    m1, r1 = _tc_pre(x, W_rel1, W_root1, b_rel1)
    p0, p1 = _seg_sum_sc(m1, src2d, dst3d)
    m2, r2 = _tc_mid(p0, p1, r1, W_rel2, W_root2, b_rel2)
    q0, q1 = _seg_sum_sc(m2, src2d, dst3d)
    out = _tc_final(q0, q1, r2)
    return out.reshape(N, 1, D)
